# baseline (speedup 1.0000x reference)
import jax
import jax.numpy as jnp
from jax import lax
from jax.experimental import pallas as pl
from jax.experimental.pallas import tpu as pltpu
from jax.experimental.pallas import tpu_sc as plsc

N = 65536
NOPS = 48
NC = 2
NS = 16
NW = NC * NS
CHUNK = N // NW
G = CHUNK // 16
D = 8


def _scatter_body(ft, iltp, opb, smt, zh, acc_out,
                  fbuf, ilbuf, sfbuf, opv,
                  idx0, idx1, idx2, idx3, val0, val1, acc_sh,
                  isem0, isem1, isem2, isem3, ssem0, ssem1):
    c = lax.axis_index("c")
    s = lax.axis_index("s")
    wid = c * NS + s
    base = wid * CHUNK
    iota16 = lax.iota(jnp.int32, 16)
    comp0 = jnp.zeros((16,), jnp.int32)
    comp1 = jnp.full((16,), 1, jnp.int32)
    comp2 = jnp.full((16,), 2, jnp.int32)
    idxs = [idx0, idx1, idx2, idx3]
    vals = [val0, val1]
    isems = [isem0, isem1, isem2, isem3]
    ssems = [ssem0, ssem1]


    pltpu.sync_copy(zh.at[pl.ds(s * 2 * CHUNK, 2 * CHUNK)],
                    acc_sh.at[pl.ds(s * 2 * CHUNK, 2 * CHUNK)])

    idx_dma = [None] * 4
    for i in range(2):
        idx_dma[i] = pltpu.async_copy(
            smt.at[pl.ds(i * N + base, CHUNK)], idxs[i], isems[i])

    for j in range(3):
        pltpu.sync_copy(ft.at[pl.ds(j * N + base, CHUNK)],
                        fbuf.at[pl.ds(j * CHUNK, CHUNK)])
    for q in range(9):
        pltpu.sync_copy(iltp.at[pl.ds(q * N + base, CHUNK)],
                        ilbuf.at[pl.ds(q * CHUNK, CHUNK)])
    pltpu.sync_copy(opb, opv)

    def sfg(g, carry):
        o = g * 16

        def pld(buf, p):
            return buf[pl.ds(p * CHUNK + o, 16)]
        fx = pld(fbuf, 0)
        fy = pld(fbuf, 1)
        fz = pld(fbuf, 2)
        for i in range(3):
            sfbuf[pl.ds(i * CHUNK + o, 16)] = (
                pld(ilbuf, i) * fx + pld(ilbuf, 3 + i) * fy + pld(ilbuf, 6 + i) * fz)
        return carry
    lax.fori_loop(0, G, sfg, None)

    plsc.subcore_barrier()

    sc_dma = [None, None]
    for i in range(NOPS):
        X = i % 2
        J = i % 4
        if sc_dma[X] is not None:
            sc_dma[X].wait()
        if i + 2 < NOPS:
            idx_dma[(i + 2) % 4] = pltpu.async_copy(
                smt.at[pl.ds((i + 2) * N + base, CHUNK)],
                idxs[(i + 2) % 4], isems[(i + 2) % 4])
        ob = i * 144
        a = [opv[pl.ds(ob + q * 16, 16)] for q in range(9)]

        def gb(g, c2, _a=a, _v=vals[X]):
            for k in range(4):
                o = g * 64 + k * 16
                sx = sfbuf[pl.ds(o, 16)]
                sy = sfbuf[pl.ds(CHUNK + o, 16)]
                sz = sfbuf[pl.ds(2 * CHUNK + o, 16)]
                rows = o + iota16
                plsc.store_scatter(_v, [rows, comp0],
                                   _a[0] * sx + _a[1] * sy + _a[2] * sz)
                plsc.store_scatter(_v, [rows, comp1],
                                   _a[3] * sx + _a[4] * sy + _a[5] * sz)
                plsc.store_scatter(_v, [rows, comp2],
                                   _a[6] * sx + _a[7] * sy + _a[8] * sz)
            return c2
        lax.fori_loop(0, G // 4, gb, None)

        idx_dma[J].wait()
        sc_dma[X] = pltpu.async_copy(vals[X], acc_sh.at[idxs[J]], ssems[X],
                                     add=True)
    sc_dma[0].wait()
    sc_dma[1].wait()

    plsc.subcore_barrier()
    pltpu.sync_copy(acc_sh.at[pl.ds(s * 2 * CHUNK, 2 * CHUNK)],
                    acc_out.at[pl.ds(c * N + s * 2 * CHUNK, 2 * CHUNK)])


def _final_body(accf, ltp, outp, a0f, a1f, lbuf, obuf):
    c = lax.axis_index("c")
    s = lax.axis_index("s")
    wid = c * NS + s
    base = wid * CHUNK
    iota16 = lax.iota(jnp.int32, 16)

    pltpu.sync_copy(accf.at[pl.ds(base * D, CHUNK * D)], a0f)
    pltpu.sync_copy(accf.at[pl.ds((N + base) * D, CHUNK * D)], a1f)
    for q in range(9):
        pltpu.sync_copy(ltp.at[pl.ds(q * N + base, CHUNK)],
                        lbuf.at[pl.ds(q * CHUNK, CHUNK)])

    def gb(g, carry):
        o = g * 16
        rows8 = (o + iota16) * D
        x = plsc.load_gather(a0f, [rows8]) + plsc.load_gather(a1f, [rows8])
        y = plsc.load_gather(a0f, [rows8 + 1]) + plsc.load_gather(a1f, [rows8 + 1])
        z = plsc.load_gather(a0f, [rows8 + 2]) + plsc.load_gather(a1f, [rows8 + 2])

        def pll(p):
            return lbuf[pl.ds(p * CHUNK + o, 16)]
        for i in range(3):
            obuf[pl.ds(i * CHUNK + o, 16)] = (
                pll(i) * x + pll(3 + i) * y + pll(6 + i) * z)
        return carry
    lax.fori_loop(0, G, gb, None)

    for i in range(3):
        pltpu.sync_copy(obuf.at[pl.ds(i * CHUNK, CHUNK)],
                        outp.at[pl.ds(i * N + base, CHUNK)])


_MESH = plsc.VectorSubcoreMesh(core_axis_name="c", subcore_axis_name="s",
                               num_cores=NC, num_subcores=NS)
_CP = pltpu.CompilerParams(needs_layout_passes=False, use_tc_tiling_on_sc=False)

_scatter_call = pl.kernel(
    _scatter_body,
    out_type=jax.ShapeDtypeStruct((NC * N, D), jnp.float32),
    compiler_params=_CP,
    mesh=_MESH,
    scratch_types=[
        pltpu.VMEM((3 * CHUNK,), jnp.float32),
        pltpu.VMEM((9 * CHUNK,), jnp.float32),
        pltpu.VMEM((3 * CHUNK,), jnp.float32),
        pltpu.VMEM((NOPS * 9 * 16,), jnp.float32),
        pltpu.VMEM((CHUNK,), jnp.int32),
        pltpu.VMEM((CHUNK,), jnp.int32),
        pltpu.VMEM((CHUNK,), jnp.int32),
        pltpu.VMEM((CHUNK,), jnp.int32),
        pltpu.VMEM((CHUNK, D), jnp.float32),
        pltpu.VMEM((CHUNK, D), jnp.float32),
        pltpu.VMEM_SHARED((N, D), jnp.float32),
        pltpu.SemaphoreType.DMA,
        pltpu.SemaphoreType.DMA,
        pltpu.SemaphoreType.DMA,
        pltpu.SemaphoreType.DMA,
        pltpu.SemaphoreType.DMA,
        pltpu.SemaphoreType.DMA,
    ],
)

_final_call = pl.kernel(
    _final_body,
    out_type=jax.ShapeDtypeStruct((3 * N,), jnp.float32),
    compiler_params=_CP,
    mesh=_MESH,
    scratch_types=[
        pltpu.VMEM((CHUNK * D,), jnp.float32),
        pltpu.VMEM((CHUNK * D,), jnp.float32),
        pltpu.VMEM((9 * CHUNK,), jnp.float32),
        pltpu.VMEM((3 * CHUNK,), jnp.float32),
    ],
)


def kernel(lattices, inv_lattices, forces, num_atoms, general_ops,
           symm_map, num_general_ops):
    ft = forces.T.reshape(-1)
    iltp = jnp.transpose(inv_lattices, (1, 2, 0)).reshape(-1)
    ltp = jnp.transpose(lattices, (1, 2, 0)).reshape(-1)
    opb = jnp.broadcast_to(general_ops.reshape(NOPS * 9, 1),
                           (NOPS * 9, 16)).reshape(-1)
    smt = symm_map.T.reshape(-1)
    zh = jnp.zeros((N, D), jnp.float32)
    acc = _scatter_call(ft, iltp, opb, smt, zh)
    outp = _final_call(acc.reshape(-1), ltp)
    return outp.reshape(3, N).T


# speedup vs baseline: 1.0853x; 1.0853x over previous
import jax
import jax.numpy as jnp
from jax import lax
from jax.experimental import pallas as pl
from jax.experimental.pallas import tpu as pltpu
from jax.experimental.pallas import tpu_sc as plsc

N = 65536
NOPS = 48
NC = 2
NS = 16
NW = NC * NS
CHUNK = N // NW
G = CHUNK // 16
D = 8


def _scatter_body(ft, iltp, opb, smt, zh, acc_out,
                  fbuf, ilbuf, sfbuf, opv,
                  idx0, idx1, val0, val1, acc_sh,
                  isem0, isem1, ssem0, ssem1):
    c = lax.axis_index("c")
    s = lax.axis_index("s")
    wid = c * NS + s
    base = wid * CHUNK
    iota16 = lax.iota(jnp.int32, 16)
    comp0 = jnp.zeros((16,), jnp.int32)
    comp1 = jnp.full((16,), 1, jnp.int32)
    comp2 = jnp.full((16,), 2, jnp.int32)


    pltpu.sync_copy(zh.at[pl.ds(s * 2 * CHUNK, 2 * CHUNK)],
                    acc_sh.at[pl.ds(s * 2 * CHUNK, 2 * CHUNK)])

    for j in range(3):
        pltpu.sync_copy(ft.at[pl.ds(j * N + base, CHUNK)],
                        fbuf.at[pl.ds(j * CHUNK, CHUNK)])
    for q in range(9):
        pltpu.sync_copy(iltp.at[pl.ds(q * N + base, CHUNK)],
                        ilbuf.at[pl.ds(q * CHUNK, CHUNK)])
    pltpu.sync_copy(opb, opv)

    def sfg(g, carry):
        o = g * 16

        def pld(buf, p):
            return buf[pl.ds(p * CHUNK + o, 16)]
        fx = pld(fbuf, 0)
        fy = pld(fbuf, 1)
        fz = pld(fbuf, 2)
        for i in range(3):
            sfbuf[pl.ds(i * CHUNK + o, 16)] = (
                pld(ilbuf, i) * fx + pld(ilbuf, 3 + i) * fy + pld(ilbuf, 6 + i) * fz)
        return carry
    lax.fori_loop(0, G, sfg, None)

    plsc.subcore_barrier()

    def compute_val(i, v):
        ob = i * 144
        a = [opv[pl.ds(ob + q * 16, 16)] for q in range(9)]

        def gb(g, c2):
            for k in range(2):
                o = g * 32 + k * 16
                sx = sfbuf[pl.ds(o, 16)]
                sy = sfbuf[pl.ds(CHUNK + o, 16)]
                sz = sfbuf[pl.ds(2 * CHUNK + o, 16)]
                rows = o + iota16
                plsc.store_scatter(v, [rows, comp0],
                                   a[0] * sx + a[1] * sy + a[2] * sz)
                plsc.store_scatter(v, [rows, comp1],
                                   a[3] * sx + a[4] * sy + a[5] * sz)
                plsc.store_scatter(v, [rows, comp2],
                                   a[6] * sx + a[7] * sy + a[8] * sz)
            return c2
        lax.fori_loop(0, G // 2, gb, None)

    def fetch_idx(i, slot, sem):
        return pltpu.async_copy(smt.at[pl.ds(i * N + base, CHUNK)],
                                slot, sem)

    d = fetch_idx(0, idx0, isem0)
    compute_val(0, val0)
    d.wait()
    pltpu.async_copy(val0, acc_sh.at[idx0], ssem0, add=True)
    d = fetch_idx(1, idx1, isem1)
    compute_val(1, val1)
    d.wait()
    pltpu.async_copy(val1, acc_sh.at[idx1], ssem1, add=True)

    def pair_body(k, carry):
        i0 = 2 * k
        pltpu.make_async_copy(val0, acc_sh.at[idx0], ssem0).wait()
        fetch_idx(i0, idx0, isem0)
        compute_val(i0, val0)
        pltpu.make_async_copy(smt.at[pl.ds(i0 * N + base, CHUNK)],
                              idx0, isem0).wait()
        pltpu.async_copy(val0, acc_sh.at[idx0], ssem0, add=True)
        i1 = i0 + 1
        pltpu.make_async_copy(val1, acc_sh.at[idx1], ssem1).wait()
        fetch_idx(i1, idx1, isem1)
        compute_val(i1, val1)
        pltpu.make_async_copy(smt.at[pl.ds(i1 * N + base, CHUNK)],
                              idx1, isem1).wait()
        pltpu.async_copy(val1, acc_sh.at[idx1], ssem1, add=True)
        return carry
    lax.fori_loop(1, NOPS // 2, pair_body, None)

    pltpu.make_async_copy(val0, acc_sh.at[idx0], ssem0).wait()
    pltpu.make_async_copy(val1, acc_sh.at[idx1], ssem1).wait()

    plsc.subcore_barrier()
    pltpu.sync_copy(acc_sh.at[pl.ds(s * 2 * CHUNK, 2 * CHUNK)],
                    acc_out.at[pl.ds(c * N + s * 2 * CHUNK, 2 * CHUNK)])


def _final_body(accf, ltp, outp, a0f, a1f, lbuf, obuf):
    c = lax.axis_index("c")
    s = lax.axis_index("s")
    wid = c * NS + s
    base = wid * CHUNK
    iota16 = lax.iota(jnp.int32, 16)

    pltpu.sync_copy(accf.at[pl.ds(base * D, CHUNK * D)], a0f)
    pltpu.sync_copy(accf.at[pl.ds((N + base) * D, CHUNK * D)], a1f)
    for q in range(9):
        pltpu.sync_copy(ltp.at[pl.ds(q * N + base, CHUNK)],
                        lbuf.at[pl.ds(q * CHUNK, CHUNK)])

    def gb(g, carry):
        o = g * 16
        rows8 = (o + iota16) * D
        x = plsc.load_gather(a0f, [rows8]) + plsc.load_gather(a1f, [rows8])
        y = plsc.load_gather(a0f, [rows8 + 1]) + plsc.load_gather(a1f, [rows8 + 1])
        z = plsc.load_gather(a0f, [rows8 + 2]) + plsc.load_gather(a1f, [rows8 + 2])

        def pll(p):
            return lbuf[pl.ds(p * CHUNK + o, 16)]
        for i in range(3):
            obuf[pl.ds(i * CHUNK + o, 16)] = (
                pll(i) * x + pll(3 + i) * y + pll(6 + i) * z)
        return carry
    lax.fori_loop(0, G, gb, None)

    for i in range(3):
        pltpu.sync_copy(obuf.at[pl.ds(i * CHUNK, CHUNK)],
                        outp.at[pl.ds(i * N + base, CHUNK)])


_MESH = plsc.VectorSubcoreMesh(core_axis_name="c", subcore_axis_name="s",
                               num_cores=NC, num_subcores=NS)
_CP = pltpu.CompilerParams(needs_layout_passes=False, use_tc_tiling_on_sc=False)

_scatter_call = pl.kernel(
    _scatter_body,
    out_type=jax.ShapeDtypeStruct((NC * N, D), jnp.float32),
    compiler_params=_CP,
    mesh=_MESH,
    scratch_types=[
        pltpu.VMEM((3 * CHUNK,), jnp.float32),
        pltpu.VMEM((9 * CHUNK,), jnp.float32),
        pltpu.VMEM((3 * CHUNK,), jnp.float32),
        pltpu.VMEM((NOPS * 9 * 16,), jnp.float32),
        pltpu.VMEM((CHUNK,), jnp.int32),
        pltpu.VMEM((CHUNK,), jnp.int32),
        pltpu.VMEM((CHUNK, D), jnp.float32),
        pltpu.VMEM((CHUNK, D), jnp.float32),
        pltpu.VMEM_SHARED((N, D), jnp.float32),
        pltpu.SemaphoreType.DMA,
        pltpu.SemaphoreType.DMA,
        pltpu.SemaphoreType.DMA,
        pltpu.SemaphoreType.DMA,
    ],
)

_final_call = pl.kernel(
    _final_body,
    out_type=jax.ShapeDtypeStruct((3 * N,), jnp.float32),
    compiler_params=_CP,
    mesh=_MESH,
    scratch_types=[
        pltpu.VMEM((CHUNK * D,), jnp.float32),
        pltpu.VMEM((CHUNK * D,), jnp.float32),
        pltpu.VMEM((9 * CHUNK,), jnp.float32),
        pltpu.VMEM((3 * CHUNK,), jnp.float32),
    ],
)


def kernel(lattices, inv_lattices, forces, num_atoms, general_ops,
           symm_map, num_general_ops):
    ft = forces.T.reshape(-1)
    iltp = jnp.transpose(inv_lattices, (1, 2, 0)).reshape(-1)
    ltp = jnp.transpose(lattices, (1, 2, 0)).reshape(-1)
    opb = jnp.broadcast_to(general_ops.reshape(NOPS * 9, 1),
                           (NOPS * 9, 16)).reshape(-1)
    smt = symm_map.T.reshape(-1)
    zh = jnp.zeros((N, D), jnp.float32)
    acc = _scatter_call(ft, iltp, opb, smt, zh)
    outp = _final_call(acc.reshape(-1), ltp)
    return outp.reshape(3, N).T
